# baseline (device time: 9673 ns/iter reference)
import jax
import jax.numpy as jnp
from jax import lax
from jax.experimental import pallas as pl
from jax.experimental.pallas import tpu as pltpu

CHUNK_ROWS = (32, 64, 128, 128, 64, 32, 32, 32)
N_CHUNKS = len(CHUNK_ROWS)


def kernel(A, B):
    A = A.astype(jnp.bfloat16)
    B = B.astype(jnp.bfloat16)
    m, k = A.shape
    k2, n = B.shape
    assert k == k2
    assert sum(CHUNK_ROWS) == m
    offs = [0]
    for r in CHUNK_ROWS:
        offs.append(offs[-1] + r)

    def body(a_ref, b_ref, out_ref, send_q, recv_q, send_s, recv_s, part_ref,
             q_send_sems, q_recv_sems, s_send_sems, s_recv_sems):
        my_x = lax.axis_index("x")
        my_y = lax.axis_index("y")
        nbr = (my_x, 1 - my_y)

        barrier_sem = pltpu.get_barrier_semaphore()
        pl.semaphore_signal(
            barrier_sem, inc=1, device_id=nbr,
            device_id_type=pl.DeviceIdType.MESH,
        )

        b = b_ref[...]

        def compute_chunk(c):
            sl = pl.ds(offs[c], CHUNK_ROWS[c])
            part = jnp.dot(a_ref[sl, :], b, preferred_element_type=jnp.float32)
            scale = jnp.max(jnp.abs(part)) + 1e-20
            send_q[sl, :] = jnp.rint(part * (127.0 / scale)).astype(jnp.int8)
            send_s[c, :, :] = (scale / 127.0) * jnp.ones(
                (8, 128), jnp.float32
            )
            part_ref[sl, :] = part.astype(jnp.bfloat16)

        compute_chunk(0)
        pl.semaphore_wait(barrier_sem, 1)

        rdmas = []
        for c in range(N_CHUNKS):
            if c > 0:
                compute_chunk(c)
            sl = pl.ds(offs[c], CHUNK_ROWS[c])
            d = pltpu.make_async_remote_copy(
                src_ref=send_q.at[sl, :],
                dst_ref=recv_q.at[sl, :],
                send_sem=q_send_sems.at[c],
                recv_sem=q_recv_sems.at[c],
                device_id=nbr,
                device_id_type=pl.DeviceIdType.MESH,
            )
            s = pltpu.make_async_remote_copy(
                src_ref=send_s.at[c],
                dst_ref=recv_s.at[c],
                send_sem=s_send_sems.at[c],
                recv_sem=s_recv_sems.at[c],
                device_id=nbr,
                device_id_type=pl.DeviceIdType.MESH,
            )
            d.start()
            s.start()
            rdmas.append((d, s))

        for c in range(N_CHUNKS):
            sl = pl.ds(offs[c], CHUNK_ROWS[c])
            d, s = rdmas[c]
            s.wait_recv()
            d.wait_recv()
            nbr_scale = recv_s[c, :1, :1]
            out_ref[sl, :] = part_ref[sl, :] + (
                recv_q[sl, :].astype(jnp.float32) * nbr_scale
            ).astype(jnp.bfloat16)

        for d, s in rdmas:
            d.wait_send()
            s.wait_send()

    return pl.pallas_call(
        body,
        out_shape=jax.ShapeDtypeStruct((m, n), jnp.bfloat16),
        in_specs=[
            pl.BlockSpec(memory_space=pltpu.VMEM),
            pl.BlockSpec(memory_space=pltpu.VMEM),
        ],
        out_specs=pl.BlockSpec(memory_space=pltpu.VMEM),
        scratch_shapes=[
            pltpu.VMEM((m, n), jnp.int8),
            pltpu.VMEM((m, n), jnp.int8),
            pltpu.VMEM((N_CHUNKS, 8, 128), jnp.float32),
            pltpu.VMEM((N_CHUNKS, 8, 128), jnp.float32),
            pltpu.VMEM((m, n), jnp.bfloat16),
            pltpu.SemaphoreType.DMA((N_CHUNKS,)),
            pltpu.SemaphoreType.DMA((N_CHUNKS,)),
            pltpu.SemaphoreType.DMA((N_CHUNKS,)),
            pltpu.SemaphoreType.DMA((N_CHUNKS,)),
        ],
        compiler_params=pltpu.CompilerParams(collective_id=0),
    )(A, B)


# device time: 9555 ns/iter; 1.0123x vs baseline; 1.0123x over previous
import jax
import jax.numpy as jnp
from jax import lax
from jax.experimental import pallas as pl
from jax.experimental.pallas import tpu as pltpu

CHUNK_ROWS = (64,) * 8
N_CHUNKS = len(CHUNK_ROWS)


def kernel(A, B):
    A = A.astype(jnp.bfloat16)
    B = B.astype(jnp.bfloat16)
    m, k = A.shape
    k2, n = B.shape
    assert k == k2
    assert sum(CHUNK_ROWS) == m
    offs = [0]
    for r in CHUNK_ROWS:
        offs.append(offs[-1] + r)

    def body(a_ref, b_ref, out_ref, send_q, recv_q, send_s, recv_s, part_ref,
             q_send_sems, q_recv_sems, s_send_sems, s_recv_sems):
        my_x = lax.axis_index("x")
        my_y = lax.axis_index("y")
        nbr = (my_x, 1 - my_y)

        barrier_sem = pltpu.get_barrier_semaphore()
        pl.semaphore_signal(
            barrier_sem, inc=1, device_id=nbr,
            device_id_type=pl.DeviceIdType.MESH,
        )

        b = b_ref[...]

        def compute_chunk(c):
            sl = pl.ds(offs[c], CHUNK_ROWS[c])
            part = jnp.dot(a_ref[sl, :], b, preferred_element_type=jnp.float32)
            scale = jnp.max(jnp.abs(part)) + 1e-20
            send_q[sl, :] = jnp.rint(part * (127.0 / scale)).astype(jnp.int8)
            send_s[c, :, :] = (scale / 127.0) * jnp.ones(
                (8, 128), jnp.float32
            )
            part_ref[sl, :] = part.astype(jnp.bfloat16)

        compute_chunk(0)
        pl.semaphore_wait(barrier_sem, 1)

        rdmas = []
        for c in range(N_CHUNKS):
            if c > 0:
                compute_chunk(c)
            sl = pl.ds(offs[c], CHUNK_ROWS[c])
            d = pltpu.make_async_remote_copy(
                src_ref=send_q.at[sl, :],
                dst_ref=recv_q.at[sl, :],
                send_sem=q_send_sems.at[c],
                recv_sem=q_recv_sems.at[c],
                device_id=nbr,
                device_id_type=pl.DeviceIdType.MESH,
            )
            s = pltpu.make_async_remote_copy(
                src_ref=send_s.at[c],
                dst_ref=recv_s.at[c],
                send_sem=s_send_sems.at[c],
                recv_sem=s_recv_sems.at[c],
                device_id=nbr,
                device_id_type=pl.DeviceIdType.MESH,
            )
            d.start()
            s.start()
            rdmas.append((d, s))

        for c in range(N_CHUNKS):
            sl = pl.ds(offs[c], CHUNK_ROWS[c])
            d, s = rdmas[c]
            s.wait_recv()
            d.wait_recv()
            nbr_scale = recv_s[c, :1, :1]
            out_ref[sl, :] = part_ref[sl, :] + (
                recv_q[sl, :].astype(jnp.float32) * nbr_scale
            ).astype(jnp.bfloat16)

        for d, s in rdmas:
            d.wait_send()
            s.wait_send()

    return pl.pallas_call(
        body,
        out_shape=jax.ShapeDtypeStruct((m, n), jnp.bfloat16),
        in_specs=[
            pl.BlockSpec(memory_space=pltpu.VMEM),
            pl.BlockSpec(memory_space=pltpu.VMEM),
        ],
        out_specs=pl.BlockSpec(memory_space=pltpu.VMEM),
        scratch_shapes=[
            pltpu.VMEM((m, n), jnp.int8),
            pltpu.VMEM((m, n), jnp.int8),
            pltpu.VMEM((N_CHUNKS, 8, 128), jnp.float32),
            pltpu.VMEM((N_CHUNKS, 8, 128), jnp.float32),
            pltpu.VMEM((m, n), jnp.bfloat16),
            pltpu.SemaphoreType.DMA((N_CHUNKS,)),
            pltpu.SemaphoreType.DMA((N_CHUNKS,)),
            pltpu.SemaphoreType.DMA((N_CHUNKS,)),
            pltpu.SemaphoreType.DMA((N_CHUNKS,)),
        ],
        compiler_params=pltpu.CompilerParams(collective_id=0),
    )(A, B)
